# baseline (speedup 1.0000x reference)
import functools

import jax
import jax.numpy as jnp
from jax import lax
from jax.experimental import pallas as pl
from jax.experimental.pallas import tpu as pltpu
from jax.experimental.pallas import tpu_sc as plsc

C = 1024
NH = 16
HD = C // NH
E = 64
TOPK = 8
EH = 256
SH_HID = 512
EPS = 1e-6
VOCAB = 32768

BT = 256
PN = 32768
NBLK = PN // BT


def _sc_gather(table, idx):
    n = idx.shape[0]
    d = table.shape[1]
    info = plsc.get_sparse_core_info()
    nw = info.num_cores * info.num_subcores
    b_per_w = n // nw
    chunk = min(b_per_w, 64)
    nchunk = b_per_w // chunk
    mesh = plsc.VectorSubcoreMesh(core_axis_name="c", subcore_axis_name="s")

    @functools.partial(
        pl.kernel,
        mesh=mesh,
        out_type=jax.ShapeDtypeStruct((n, d), jnp.float32),
        scratch_types=[
            pltpu.VMEM((chunk,), jnp.int32),
            pltpu.VMEM((chunk, d), jnp.float32),
            pltpu.SemaphoreType.DMA,
        ],
    )
    def k(table_hbm, idx_hbm, out_hbm, idx_c, rows_v, sem):
        wid = lax.axis_index("s") * info.num_cores + lax.axis_index("c")
        base = wid * b_per_w
        for c in range(nchunk):
            r0 = base + c * chunk
            pltpu.sync_copy(idx_hbm.at[pl.ds(r0, chunk)], idx_c)
            pltpu.async_copy(table_hbm.at[idx_c], rows_v, sem).wait()
            pltpu.sync_copy(rows_v, out_hbm.at[pl.ds(r0, chunk)])

    return k(table, idx)


def _rms(x, w):
    ms = jnp.mean(jnp.square(x), axis=-1, keepdims=True)
    return x / jnp.sqrt(ms + EPS) * w


def _silu(x):
    return x * jax.nn.sigmoid(x)


def _qkv_body(x_ref, w_ref, ln_ref, qn_ref, kn_ref, sin_ref, cos_ref,
              q_ref, k_ref, v_ref):
    h = _rms(x_ref[...], ln_ref[0])
    qkv = lax.dot_general(h, w_ref[...], (((1,), (1,)), ((), ())))
    bt = qkv.shape[0]
    q = qkv[:, :C].reshape(bt, NH, HD)
    k = qkv[:, C:2 * C].reshape(bt, NH, HD)
    v = qkv[:, 2 * C:]
    s = sin_ref[...][:, None, :]
    c = cos_ref[...][:, None, :]

    def rope(z):
        z1 = z[..., :HD // 2]
        z2 = z[..., HD // 2:]
        return jnp.concatenate([z1 * c - z2 * s, z1 * s + z2 * c], axis=-1)

    q = _rms(rope(q), qn_ref[0])
    k = _rms(rope(k), kn_ref[0])
    q_ref[...] = jnp.transpose(q, (1, 0, 2))
    k_ref[...] = jnp.transpose(k, (1, 0, 2))
    v_ref[...] = jnp.transpose(v.reshape(bt, NH, HD), (1, 0, 2))


def _qkv(x, lp, sin, cos):
    t = x.shape[0]
    btok = 256
    grid = (t // btok,)
    return pl.pallas_call(
        _qkv_body,
        grid=grid,
        in_specs=[
            pl.BlockSpec((btok, C), lambda i: (i, 0)),
            pl.BlockSpec((3 * C, C), lambda i: (0, 0)),
            pl.BlockSpec((1, C), lambda i: (0, 0)),
            pl.BlockSpec((1, HD), lambda i: (0, 0)),
            pl.BlockSpec((1, HD), lambda i: (0, 0)),
            pl.BlockSpec((btok, HD // 2), lambda i: (i, 0)),
            pl.BlockSpec((btok, HD // 2), lambda i: (i, 0)),
        ],
        out_specs=[
            pl.BlockSpec((NH, btok, HD), lambda i: (0, i, 0)),
            pl.BlockSpec((NH, btok, HD), lambda i: (0, i, 0)),
            pl.BlockSpec((NH, btok, HD), lambda i: (0, i, 0)),
        ],
        out_shape=[jax.ShapeDtypeStruct((NH, t, HD), jnp.float32)] * 3,
    )(x, lp['attn_w'], lp['ln1_w'].reshape(1, C),
      lp['q_norm_w'].reshape(1, HD), lp['k_norm_w'].reshape(1, HD), sin, cos)


def _attn_body(q_ref, k_ref, v_ref, o_ref, acc_ref, m_ref, l_ref):
    qi = pl.program_id(1)
    kj = pl.program_id(2)
    npj = pl.num_programs(2)

    @pl.when(kj == 0)
    def _init():
        acc_ref[...] = jnp.zeros_like(acc_ref)
        m_ref[...] = jnp.full_like(m_ref, -1e30)
        l_ref[...] = jnp.zeros_like(l_ref)

    @pl.when(kj <= qi)
    def _compute():
        q = q_ref[0]
        k = k_ref[0]
        v = v_ref[0]
        bq, bk = q.shape[0], k.shape[0]
        s = lax.dot_general(q, k, (((1,), (1,)), ((), ()))) * (1.0 / 8.0)
        rows = lax.broadcasted_iota(jnp.int32, (bq, bk), 0)
        cols = lax.broadcasted_iota(jnp.int32, (bq, bk), 1)
        s = jnp.where((kj < qi) | (cols <= rows), s, -1e30)
        m_prev = m_ref[:, :1]
        m_new = jnp.maximum(m_prev, jnp.max(s, axis=-1, keepdims=True))
        alpha = jnp.exp(m_prev - m_new)
        p = jnp.exp(s - m_new)
        l_ref[:, :1] = l_ref[:, :1] * alpha + jnp.sum(p, axis=-1, keepdims=True)
        m_ref[:, :1] = m_new
        acc_ref[...] = acc_ref[...] * alpha + jnp.dot(p, v)

    @pl.when(kj == npj - 1)
    def _fin():
        o_ref[0] = acc_ref[...] / l_ref[:, :1]


def _attention(q, k, v):
    t = q.shape[1]
    bq = 256
    bk = 256
    grid = (NH, t // bq, t // bk)
    return pl.pallas_call(
        _attn_body,
        grid=grid,
        in_specs=[
            pl.BlockSpec((1, bq, HD), lambda h, i, j: (h, i, 0)),
            pl.BlockSpec((1, bk, HD), lambda h, i, j: (h, j, 0)),
            pl.BlockSpec((1, bk, HD), lambda h, i, j: (h, j, 0)),
        ],
        out_specs=pl.BlockSpec((1, bq, HD), lambda h, i, j: (h, i, 0)),
        out_shape=jax.ShapeDtypeStruct((NH, t, HD), jnp.float32),
        scratch_shapes=[
            pltpu.VMEM((bq, HD), jnp.float32),
            pltpu.VMEM((bq, 1), jnp.float32),
            pltpu.VMEM((bq, 1), jnp.float32),
        ],
    )(q, k, v)


def _proj_body(a_ref, w_ref, x_ref, ln_ref, xn_ref, h2_ref):
    bt = a_ref.shape[1]
    am = jnp.transpose(a_ref[...], (1, 0, 2)).reshape(bt, C)
    a = lax.dot_general(am, w_ref[...], (((1,), (1,)), ((), ())))
    xn = x_ref[...] + a
    xn_ref[...] = xn
    h2_ref[...] = _rms(xn, ln_ref[0])


def _proj(attn, x, lp):
    t = x.shape[0]
    btok = 256
    return pl.pallas_call(
        _proj_body,
        grid=(t // btok,),
        in_specs=[
            pl.BlockSpec((NH, btok, HD), lambda i: (0, i, 0)),
            pl.BlockSpec((C, C), lambda i: (0, 0)),
            pl.BlockSpec((btok, C), lambda i: (i, 0)),
            pl.BlockSpec((1, C), lambda i: (0, 0)),
        ],
        out_specs=[
            pl.BlockSpec((btok, C), lambda i: (i, 0)),
            pl.BlockSpec((btok, C), lambda i: (i, 0)),
        ],
        out_shape=[jax.ShapeDtypeStruct((t, C), jnp.float32)] * 2,
    )(attn, lp['proj_w'], x, lp['ln2_w'].reshape(1, C))


def _router_body(h2_ref, gw_ref, gb_ref, ti_ref, wn_ref):
    s = jax.nn.sigmoid(
        lax.dot_general(h2_ref[...], gw_ref[...], (((1,), (1,)), ((), ()))))
    work = s + gb_ref[0]
    bt = s.shape[0]
    iota = lax.broadcasted_iota(jnp.int32, (bt, E), 1)
    idxs = []
    wvals = []
    for _ in range(TOPK):
        m = jnp.max(work, axis=-1, keepdims=True)
        cand = jnp.where(work == m, iota, E)
        ix = jnp.min(cand, axis=-1, keepdims=True)
        hit = iota == ix
        wvals.append(jnp.sum(jnp.where(hit, s, 0.0), axis=-1, keepdims=True))
        idxs.append(ix)
        work = jnp.where(hit, -jnp.float32(1e30), work)
    ti_ref[...] = jnp.concatenate(idxs, axis=-1)
    wv = jnp.concatenate(wvals, axis=-1)
    wn_ref[...] = wv / jnp.sum(wv, axis=-1, keepdims=True)


def _router(h2, lp):
    t = h2.shape[0]
    btok = 256
    return pl.pallas_call(
        _router_body,
        grid=(t // btok,),
        in_specs=[
            pl.BlockSpec((btok, C), lambda i: (i, 0)),
            pl.BlockSpec((E, C), lambda i: (0, 0)),
            pl.BlockSpec((1, E), lambda i: (0, 0)),
        ],
        out_specs=[
            pl.BlockSpec((btok, TOPK), lambda i: (i, 0)),
            pl.BlockSpec((btok, TOPK), lambda i: (i, 0)),
        ],
        out_shape=[
            jax.ShapeDtypeStruct((t, TOPK), jnp.int32),
            jax.ShapeDtypeStruct((t, TOPK), jnp.float32),
        ],
    )(h2, lp['gate_w'], lp['gate_bias'].reshape(1, E))


def _moe_body(be_ref, xs_ref, up_ref, dn_ref, ws_ref, o_ref):
    b = pl.program_id(0)

    @pl.when(b < be_ref[NBLK])
    def _go():
        x = xs_ref[...]
        h = lax.dot_general(x, up_ref[0], (((1,), (1,)), ((), ())))
        hs = _silu(h[:, :EH]) * h[:, EH:]
        d = lax.dot_general(hs, dn_ref[0], (((1,), (1,)), ((), ())))
        o_ref[...] = d * ws_ref[0, 0][:, None]


def _moe_gemm(be, xs, up_w, down_w, ws3):
    grid_spec = pltpu.PrefetchScalarGridSpec(
        num_scalar_prefetch=1,
        grid=(NBLK,),
        in_specs=[
            pl.BlockSpec((BT, C), lambda b, be: (b, 0)),
            pl.BlockSpec((1, 2 * EH, C), lambda b, be: (be[b], 0, 0)),
            pl.BlockSpec((1, C, EH), lambda b, be: (be[b], 0, 0)),
            pl.BlockSpec((1, 1, BT), lambda b, be: (b, 0, 0)),
        ],
        out_specs=pl.BlockSpec((BT, C), lambda b, be: (b, 0)),
    )
    return pl.pallas_call(
        _moe_body,
        grid_spec=grid_spec,
        out_shape=jax.ShapeDtypeStruct((PN, C), jnp.float32),
    )(be, xs, up_w, down_w, ws3)


def _comb_body(op_ref, h2_ref, x_ref, sgw_ref, sdw_ref, o_ref):
    moe = op_ref[:, :C]
    for k in range(1, TOPK):
        moe = moe + op_ref[:, k * C:(k + 1) * C]
    gp = lax.dot_general(h2_ref[...], sgw_ref[...], (((1,), (1,)), ((), ())))
    y = gp[:, :SH_HID]
    g = gp[:, SH_HID:]
    sh = lax.dot_general(_silu(g) * y, sdw_ref[...], (((1,), (1,)), ((), ())))
    o_ref[...] = x_ref[...] + sh + moe


def _combine(op, h2, x, lp):
    t = x.shape[0]
    btok = 256
    return pl.pallas_call(
        _comb_body,
        grid=(t // btok,),
        in_specs=[
            pl.BlockSpec((btok, TOPK * C), lambda i: (i, 0)),
            pl.BlockSpec((btok, C), lambda i: (i, 0)),
            pl.BlockSpec((btok, C), lambda i: (i, 0)),
            pl.BlockSpec((2 * SH_HID, C), lambda i: (0, 0)),
            pl.BlockSpec((C, SH_HID), lambda i: (0, 0)),
        ],
        out_specs=pl.BlockSpec((btok, C), lambda i: (i, 0)),
        out_shape=jax.ShapeDtypeStruct((t, C), jnp.float32),
    )(op, h2, x, lp['shared_gate_w'], lp['shared_down_w'])


def _head_body(x_ref, ln_ref, w_ref, o_ref):
    xn = _rms(x_ref[...], ln_ref[0])
    o_ref[...] = lax.dot_general(xn, w_ref[...], (((1,), (1,)), ((), ())))


def _lm_head(x, ln_w, wte):
    t = x.shape[0]
    btok = 256
    bv = 1024
    return pl.pallas_call(
        _head_body,
        grid=(VOCAB // bv, t // btok),
        in_specs=[
            pl.BlockSpec((btok, C), lambda j, i: (i, 0)),
            pl.BlockSpec((1, C), lambda j, i: (0, 0)),
            pl.BlockSpec((bv, C), lambda j, i: (j, 0)),
        ],
        out_specs=pl.BlockSpec((btok, bv), lambda j, i: (i, j)),
        out_shape=jax.ShapeDtypeStruct((t, VOCAB), jnp.float32),
    )(x, ln_w.reshape(1, C), wte)


def _route_tables(ti, wn):
    n = ti.shape[0] * TOPK
    eids = ti.reshape(-1)
    wflat = wn.reshape(-1)
    counts = jnp.bincount(eids, length=E)
    offs = jnp.cumsum(counts) - counts
    pe = ((counts + BT - 1) // BT) * BT
    pb = jnp.cumsum(pe) - pe
    order = jnp.argsort(eids)
    es = eids[order]
    ppos_sorted = pb[es] + (jnp.arange(n, dtype=jnp.int32) - offs[es])
    tok_padded = jnp.zeros((PN,), jnp.int32).at[ppos_sorted].set(
        (order // TOPK).astype(jnp.int32))
    ws_padded = jnp.zeros((PN,), jnp.float32).at[ppos_sorted].set(wflat[order])
    ppos = jnp.zeros((n,), jnp.int32).at[order].set(
        ppos_sorted.astype(jnp.int32))
    nb = pe // BT
    be = jnp.repeat(jnp.arange(E, dtype=jnp.int32), nb,
                    total_repeat_length=NBLK)
    be = jnp.concatenate([be, jnp.sum(nb, dtype=jnp.int32)[None]])
    return tok_padded, ws_padded.reshape(NBLK, 1, BT), ppos, be


def _layer(x, lp, sin, cos):
    q, k, v = _qkv(x, lp, sin, cos)
    attn = _attention(q, k, v)
    xn, h2 = _proj(attn, x, lp)
    ti, wn = _router(h2, lp)
    tok_padded, ws3, ppos, be = _route_tables(ti, wn)
    xs = _sc_gather(h2, tok_padded)
    out_sorted = _moe_gemm(be, xs, lp['up_w'], lp['down_w'], ws3)
    op = _sc_gather(out_sorted, ppos)
    t = x.shape[0]
    return _combine(op.reshape(t, TOPK * C), h2, xn, lp)


def kernel(params, idx):
    t = idx.shape[1]
    ids = idx.reshape(-1).astype(jnp.int32)
    x = _sc_gather(params['wte'], ids)
    inv = 1.0 / (10000.0 ** (jnp.arange(0, HD, 2, dtype=jnp.float32) / HD))
    ang = jnp.arange(t, dtype=jnp.float32)[:, None] * inv[None, :]
    sin, cos = jnp.sin(ang), jnp.cos(ang)
    for lp in params['blocks']:
        x = _layer(x, lp, sin, cos)
    logits = _lm_head(x, params['ln_w'], params['wte'])
    return logits.reshape(1, t, VOCAB)


# speedup vs baseline: 1.8915x; 1.8915x over previous
import functools

import jax
import jax.numpy as jnp
from jax import lax
from jax.experimental import pallas as pl
from jax.experimental.pallas import tpu as pltpu
from jax.experimental.pallas import tpu_sc as plsc

C = 1024
NH = 16
HD = C // NH
E = 64
TOPK = 8
EH = 256
SH_HID = 512
EPS = 1e-6
VOCAB = 32768

BT = 128
PN = 24576
NBLK = PN // BT


def _sc_gather(table, idx, nvalid=None):
    n = idx.shape[0]
    d = table.shape[1]
    dt = table.dtype
    if dt == jnp.bfloat16:
        row_shape = (d // 128, 128)
        table = table.reshape(table.shape[0], *row_shape)
    else:
        row_shape = (d,)
    info = plsc.get_sparse_core_info()
    nw = info.num_cores * info.num_subcores
    b_per_w = n // nw
    chunk = min(b_per_w, 32)
    nchunk = b_per_w // chunk
    mesh = plsc.VectorSubcoreMesh(core_axis_name="c", subcore_axis_name="s")
    limited = nvalid is not None
    args = (table, idx) + ((nvalid,) if limited else ())

    @functools.partial(
        pl.kernel,
        mesh=mesh,
        out_type=jax.ShapeDtypeStruct((n,) + row_shape, dt),
        scratch_types=[
            pltpu.VMEM((chunk,), jnp.int32),
            pltpu.VMEM((chunk,), jnp.int32),
            pltpu.VMEM((chunk,) + row_shape, dt),
            pltpu.VMEM((chunk,) + row_shape, dt),
            pltpu.VMEM((16,), jnp.int32),
            pltpu.SemaphoreType.DMA,
            pltpu.SemaphoreType.DMA,
            pltpu.SemaphoreType.DMA,
            pltpu.SemaphoreType.DMA,
        ],
    )
    def k(*refs):
        if limited:
            (table_hbm, idx_hbm, nv_hbm, out_hbm, idx0, idx1, rows0, rows1,
             nv_v, g0, g1, w0, w1) = refs
            pltpu.sync_copy(nv_hbm, nv_v)
            nvvec = nv_v[...]
        else:
            (table_hbm, idx_hbm, out_hbm, idx0, idx1, rows0, rows1,
             nv_v, g0, g1, w0, w1) = refs
        wid = lax.axis_index("s") * info.num_cores + lax.axis_index("c")
        base = wid * b_per_w
        idxb = (idx0, idx1)
        rowb = (rows0, rows1)
        gsem = (g0, g1)
        wsem = (w0, w1)
        gd = [None, None]
        wd = [None, None]

        def gated(c, fn):
            if limited:
                pl.when(jnp.all((base + c * chunk) < nvvec))(fn)
            else:
                fn()

        for c in range(nchunk):
            b = c & 1

            def issue(b=b, c=c):
                if wd[b] is not None:
                    wd[b].wait()
                r0 = pl.multiple_of(base + c * chunk, chunk)
                pltpu.sync_copy(idx_hbm.at[pl.ds(r0, chunk)], idxb[b])
                gd[b] = pltpu.async_copy(table_hbm.at[idxb[b]], rowb[b],
                                         gsem[b])

            gated(c, issue)
            if c >= 1:

                def drain(c=c):
                    pb = (c - 1) & 1
                    gd[pb].wait()
                    r1 = pl.multiple_of(base + (c - 1) * chunk, chunk)
                    wd[pb] = pltpu.async_copy(
                        rowb[pb], out_hbm.at[pl.ds(r1, chunk)], wsem[pb])

                gated(c - 1, drain)

        def drain_last():
            pb = (nchunk - 1) & 1
            gd[pb].wait()
            r2 = pl.multiple_of(base + (nchunk - 1) * chunk, chunk)
            pltpu.sync_copy(rowb[pb], out_hbm.at[pl.ds(r2, chunk)])

        gated(nchunk - 1, drain_last)
        if nchunk >= 2:

            def fin():
                wd[(nchunk - 2) & 1].wait()

            gated(nchunk - 2, fin)

    return k(*args).reshape(n, d)


def _rms(x, w):
    ms = jnp.mean(jnp.square(x), axis=-1, keepdims=True)
    return x / jnp.sqrt(ms + EPS) * w


def _silu(x):
    return x * jax.nn.sigmoid(x)


def _bdot(a, b, dims):
    return lax.dot_general(a.astype(jnp.bfloat16), b.astype(jnp.bfloat16),
                           (dims, ((), ())),
                           preferred_element_type=jnp.float32)


def _qkv_body(x_ref, w_ref, ln_ref, qn_ref, kn_ref, sin_ref, cos_ref,
              q_ref, k_ref, v_ref):
    h = _rms(x_ref[...], ln_ref[0])
    qkv = lax.dot_general(h, w_ref[...], (((1,), (1,)), ((), ())))
    bt = qkv.shape[0]
    q = qkv[:, :C].reshape(bt, NH, HD)
    k = qkv[:, C:2 * C].reshape(bt, NH, HD)
    v = qkv[:, 2 * C:]
    s = sin_ref[...][:, None, :]
    c = cos_ref[...][:, None, :]

    def rope(z):
        z1 = z[..., :HD // 2]
        z2 = z[..., HD // 2:]
        return jnp.concatenate([z1 * c - z2 * s, z1 * s + z2 * c], axis=-1)

    q = _rms(rope(q), qn_ref[0])
    k = _rms(rope(k), kn_ref[0])
    q_ref[...] = jnp.transpose(q, (1, 0, 2))
    k_ref[...] = jnp.transpose(k, (1, 0, 2))
    v_ref[...] = jnp.transpose(v.reshape(bt, NH, HD), (1, 0, 2))


def _qkv(x, lp, sin, cos):
    t = x.shape[0]
    btok = 256
    grid = (t // btok,)
    return pl.pallas_call(
        _qkv_body,
        grid=grid,
        in_specs=[
            pl.BlockSpec((btok, C), lambda i: (i, 0)),
            pl.BlockSpec((3 * C, C), lambda i: (0, 0)),
            pl.BlockSpec((1, C), lambda i: (0, 0)),
            pl.BlockSpec((1, HD), lambda i: (0, 0)),
            pl.BlockSpec((1, HD), lambda i: (0, 0)),
            pl.BlockSpec((btok, HD // 2), lambda i: (i, 0)),
            pl.BlockSpec((btok, HD // 2), lambda i: (i, 0)),
        ],
        out_specs=[
            pl.BlockSpec((NH, btok, HD), lambda i: (0, i, 0)),
            pl.BlockSpec((NH, btok, HD), lambda i: (0, i, 0)),
            pl.BlockSpec((NH, btok, HD), lambda i: (0, i, 0)),
        ],
        out_shape=[jax.ShapeDtypeStruct((NH, t, HD), jnp.float32)] * 3,
    )(x, lp['attn_w'], lp['ln1_w'].reshape(1, C),
      lp['q_norm_w'].reshape(1, HD), lp['k_norm_w'].reshape(1, HD), sin, cos)


def _attn_body(q_ref, k_ref, v_ref, o_ref):
    qi = pl.program_id(1)
    q = q_ref[0]
    k = k_ref[0]
    v = v_ref[0]
    bq, t = q.shape[0], k.shape[0]
    s = lax.dot_general(q, k, (((1,), (1,)), ((), ()))) * (1.0 / 8.0)
    rows = qi * bq + lax.broadcasted_iota(jnp.int32, (bq, t), 0)
    cols = lax.broadcasted_iota(jnp.int32, (bq, t), 1)
    s = jnp.where(cols <= rows, s, -1e30)
    m = jnp.max(s, axis=-1, keepdims=True)
    p = jnp.exp(s - m)
    w = p / jnp.sum(p, axis=-1, keepdims=True)
    o_ref[0] = jnp.dot(w, v)


def _attention(q, k, v):
    t = q.shape[1]
    bq = 256
    grid = (NH, t // bq)
    return pl.pallas_call(
        _attn_body,
        grid=grid,
        in_specs=[
            pl.BlockSpec((1, bq, HD), lambda h, i: (h, i, 0)),
            pl.BlockSpec((1, t, HD), lambda h, i: (h, 0, 0)),
            pl.BlockSpec((1, t, HD), lambda h, i: (h, 0, 0)),
        ],
        out_specs=pl.BlockSpec((1, bq, HD), lambda h, i: (h, i, 0)),
        out_shape=jax.ShapeDtypeStruct((NH, t, HD), jnp.float32),
    )(q, k, v)


def _proj_body(a_ref, w_ref, x_ref, ln_ref, xn_ref, h2_ref):
    bt = a_ref.shape[1]
    am = jnp.transpose(a_ref[...], (1, 0, 2)).reshape(bt, C)
    a = lax.dot_general(am, w_ref[...], (((1,), (1,)), ((), ())))
    xn = x_ref[...] + a
    xn_ref[...] = xn
    h2_ref[...] = _rms(xn, ln_ref[0])


def _proj(attn, x, lp):
    t = x.shape[0]
    btok = 256
    return pl.pallas_call(
        _proj_body,
        grid=(t // btok,),
        in_specs=[
            pl.BlockSpec((NH, btok, HD), lambda i: (0, i, 0)),
            pl.BlockSpec((C, C), lambda i: (0, 0)),
            pl.BlockSpec((btok, C), lambda i: (i, 0)),
            pl.BlockSpec((1, C), lambda i: (0, 0)),
        ],
        out_specs=[
            pl.BlockSpec((btok, C), lambda i: (i, 0)),
            pl.BlockSpec((btok, C), lambda i: (i, 0)),
        ],
        out_shape=[jax.ShapeDtypeStruct((t, C), jnp.float32)] * 2,
    )(attn, lp['proj_w'], x, lp['ln2_w'].reshape(1, C))


def _router_body(h2_ref, gw_ref, gb_ref, ti_ref, wn_ref):
    s = jax.nn.sigmoid(
        lax.dot_general(h2_ref[...], gw_ref[...], (((1,), (1,)), ((), ()))))
    work = s + gb_ref[0]
    bt = s.shape[0]
    iota = lax.broadcasted_iota(jnp.int32, (bt, E), 1)
    idxs = []
    wvals = []
    for _ in range(TOPK):
        m = jnp.max(work, axis=-1, keepdims=True)
        cand = jnp.where(work == m, iota, E)
        ix = jnp.min(cand, axis=-1, keepdims=True)
        hit = iota == ix
        wvals.append(jnp.sum(jnp.where(hit, s, 0.0), axis=-1, keepdims=True))
        idxs.append(ix)
        work = jnp.where(hit, -jnp.float32(1e30), work)
    ti_ref[...] = jnp.concatenate(idxs, axis=-1)
    wv = jnp.concatenate(wvals, axis=-1)
    wn_ref[...] = wv / jnp.sum(wv, axis=-1, keepdims=True)


def _router(h2, lp):
    t = h2.shape[0]
    btok = 256
    return pl.pallas_call(
        _router_body,
        grid=(t // btok,),
        in_specs=[
            pl.BlockSpec((btok, C), lambda i: (i, 0)),
            pl.BlockSpec((E, C), lambda i: (0, 0)),
            pl.BlockSpec((1, E), lambda i: (0, 0)),
        ],
        out_specs=[
            pl.BlockSpec((btok, TOPK), lambda i: (i, 0)),
            pl.BlockSpec((btok, TOPK), lambda i: (i, 0)),
        ],
        out_shape=[
            jax.ShapeDtypeStruct((t, TOPK), jnp.int32),
            jax.ShapeDtypeStruct((t, TOPK), jnp.float32),
        ],
    )(h2, lp['gate_w'], lp['gate_bias'].reshape(1, E))


def _moe_body(be_ref, xs_ref, up_ref, dn_ref, ws_ref, o_ref):
    b = pl.program_id(0)

    @pl.when(b < be_ref[NBLK])
    def _go():
        x = xs_ref[...]
        h = _bdot(x, up_ref[0], ((1,), (1,)))
        hs = _silu(h[:, :EH]) * h[:, EH:]
        d = _bdot(hs, dn_ref[0], ((1,), (1,)))
        o_ref[...] = d * ws_ref[0, 0][:, None]


def _moe_gemm(be, xs, up_w, down_w, ws3):
    grid_spec = pltpu.PrefetchScalarGridSpec(
        num_scalar_prefetch=1,
        grid=(NBLK,),
        in_specs=[
            pl.BlockSpec((BT, C), lambda b, be: (b, 0)),
            pl.BlockSpec((1, 2 * EH, C), lambda b, be: (be[b], 0, 0)),
            pl.BlockSpec((1, C, EH), lambda b, be: (be[b], 0, 0)),
            pl.BlockSpec((1, 1, BT), lambda b, be: (b, 0, 0)),
        ],
        out_specs=pl.BlockSpec((BT, C), lambda b, be: (b, 0)),
    )
    return pl.pallas_call(
        _moe_body,
        grid_spec=grid_spec,
        out_shape=jax.ShapeDtypeStruct((PN, C), jnp.float32),
    )(be, xs, up_w, down_w, ws3)


def _comb_body(op_ref, h2_ref, x_ref, sgw_ref, sdw_ref, o_ref):
    moe = op_ref[:, :C]
    for k in range(1, TOPK):
        moe = moe + op_ref[:, k * C:(k + 1) * C]
    gp = _bdot(h2_ref[...], sgw_ref[...], ((1,), (1,)))
    y = gp[:, :SH_HID]
    g = gp[:, SH_HID:]
    sh = _bdot(_silu(g) * y, sdw_ref[...], ((1,), (1,)))
    o_ref[...] = x_ref[...] + sh + moe


def _combine(op, h2, x, lp):
    t = x.shape[0]
    btok = 256
    return pl.pallas_call(
        _comb_body,
        grid=(t // btok,),
        in_specs=[
            pl.BlockSpec((btok, TOPK * C), lambda i: (i, 0)),
            pl.BlockSpec((btok, C), lambda i: (i, 0)),
            pl.BlockSpec((btok, C), lambda i: (i, 0)),
            pl.BlockSpec((2 * SH_HID, C), lambda i: (0, 0)),
            pl.BlockSpec((C, SH_HID), lambda i: (0, 0)),
        ],
        out_specs=pl.BlockSpec((btok, C), lambda i: (i, 0)),
        out_shape=jax.ShapeDtypeStruct((t, C), jnp.float32),
    )(op, h2, x, lp['shared_gate_w'], lp['shared_down_w'])


def _head_body(x_ref, ln_ref, w_ref, o_ref):
    xn = _rms(x_ref[...], ln_ref[0])
    o_ref[...] = _bdot(xn, w_ref[...], ((1,), (1,)))


def _lm_head(x, ln_w, wte):
    t = x.shape[0]
    btok = 256
    bv = 1024
    return pl.pallas_call(
        _head_body,
        grid=(VOCAB // bv, t // btok),
        in_specs=[
            pl.BlockSpec((btok, C), lambda j, i: (i, 0)),
            pl.BlockSpec((1, C), lambda j, i: (0, 0)),
            pl.BlockSpec((bv, C), lambda j, i: (j, 0)),
        ],
        out_specs=pl.BlockSpec((btok, bv), lambda j, i: (i, j)),
        out_shape=jax.ShapeDtypeStruct((t, VOCAB), jnp.float32),
    )(x, ln_w.reshape(1, C), wte)


def _route_tables(ti, wn):
    n = ti.shape[0] * TOPK
    eids = ti.reshape(-1)
    wflat = wn.reshape(-1)
    counts = jnp.bincount(eids, length=E)
    offs = jnp.cumsum(counts) - counts
    pe = ((counts + BT - 1) // BT) * BT
    pb = jnp.cumsum(pe) - pe
    order = jnp.argsort(eids)
    es = eids[order]
    ppos_sorted = pb[es] + (jnp.arange(n, dtype=jnp.int32) - offs[es])
    ntok = ti.shape[0]
    tok_padded = (jnp.arange(PN, dtype=jnp.int32) % ntok).at[ppos_sorted].set(
        (order // TOPK).astype(jnp.int32))
    ws_padded = jnp.zeros((PN,), jnp.float32).at[ppos_sorted].set(wflat[order])
    ppos = jnp.zeros((n,), jnp.int32).at[order].set(
        ppos_sorted.astype(jnp.int32))
    nb = pe // BT
    be = jnp.repeat(jnp.arange(E, dtype=jnp.int32), nb,
                    total_repeat_length=NBLK)
    nvb = jnp.sum(nb, dtype=jnp.int32)
    be = jnp.concatenate([be, nvb[None]])
    nv16 = jnp.broadcast_to(nvb * BT, (16,)).astype(jnp.int32)
    return tok_padded, ws_padded.reshape(NBLK, 1, BT), ppos, be, nv16


def _layer(x, lp, sin, cos):
    q, k, v = _qkv(x, lp, sin, cos)
    attn = _attention(q, k, v)
    xn, h2 = _proj(attn, x, lp)
    ti, wn = _router(h2, lp)
    tok_padded, ws3, ppos, be, nv16 = _route_tables(ti, wn)
    xs = _sc_gather(h2, tok_padded)
    out_sorted = _moe_gemm(be, xs, lp['up_w'], lp['down_w'], ws3)
    op = _sc_gather(out_sorted, ppos)
    t = x.shape[0]
    return _combine(op.reshape(t, TOPK * C), h2, xn, lp)


def kernel(params, idx):
    t = idx.shape[1]
    ids = idx.reshape(-1).astype(jnp.int32)
    x = _sc_gather(params['wte'], ids)
    inv = 1.0 / (10000.0 ** (jnp.arange(0, HD, 2, dtype=jnp.float32) / HD))
    ang = jnp.arange(t, dtype=jnp.float32)[:, None] * inv[None, :]
    sin, cos = jnp.sin(ang), jnp.cos(ang)
    for lp in params['blocks']:
        x = _layer(x, lp, sin, cos)
    logits = _lm_head(x, params['ln_w'], params['wte'])
    return logits.reshape(1, t, VOCAB)


# speedup vs baseline: 2.1834x; 1.1543x over previous
import functools

import jax
import jax.numpy as jnp
from jax import lax
from jax.experimental import pallas as pl
from jax.experimental.pallas import tpu as pltpu
from jax.experimental.pallas import tpu_sc as plsc

C = 1024
NH = 16
HD = C // NH
E = 64
TOPK = 8
EH = 256
SH_HID = 512
EPS = 1e-6
VOCAB = 32768

BT = 128
PN = 24576
NBLK = PN // BT


def _sc_gather(table, idx, nvalid=None):
    n = idx.shape[0]
    d = table.shape[1]
    dt = table.dtype
    if dt == jnp.bfloat16:
        row_shape = (d // 128, 128)
        table = table.reshape(table.shape[0], *row_shape)
    else:
        row_shape = (d,)
    info = plsc.get_sparse_core_info()
    nw = info.num_cores * info.num_subcores
    b_per_w = n // nw
    chunk = min(b_per_w, 32)
    nchunk = b_per_w // chunk
    mesh = plsc.VectorSubcoreMesh(core_axis_name="c", subcore_axis_name="s")
    args = (table, idx)

    NBUF = 3

    @functools.partial(
        pl.kernel,
        mesh=mesh,
        out_type=jax.ShapeDtypeStruct((n,) + row_shape, dt),
        scratch_types=(
            [pltpu.VMEM((chunk,), jnp.int32) for _ in range(NBUF)]
            + [pltpu.VMEM((chunk,) + row_shape, dt) for _ in range(NBUF)]
            + [pltpu.SemaphoreType.DMA for _ in range(2 * NBUF)]
        ),
    )
    def k(*refs):
        table_hbm, idx_hbm = refs[0], refs[1]
        out_hbm = refs[2]
        sc = refs[3:]
        idxb = sc[:NBUF]
        rowb = sc[NBUF:2 * NBUF]
        gsem = sc[2 * NBUF:3 * NBUF]
        wsem = sc[3 * NBUF:4 * NBUF]
        wid = lax.axis_index("s") * info.num_cores + lax.axis_index("c")
        base = wid * b_per_w
        gd = [None] * NBUF
        wd = [None] * NBUF

        def write_out(d):
            pb = d % NBUF
            gd[pb].wait()
            wd[pb] = pltpu.async_copy(
                rowb[pb], out_hbm.at[pl.ds(base + d * chunk, chunk)],
                wsem[pb])

        for c in range(nchunk):
            b = c % NBUF
            if wd[b] is not None:
                wd[b].wait()
            pltpu.sync_copy(idx_hbm.at[pl.ds(base + c * chunk, chunk)],
                            idxb[b])
            gd[b] = pltpu.async_copy(table_hbm.at[idxb[b]], rowb[b], gsem[b])
            if c >= NBUF - 1:
                write_out(c - (NBUF - 1))
        for d in range(max(0, nchunk - (NBUF - 1)), nchunk):
            write_out(d)
        for b in range(min(NBUF, nchunk)):
            if wd[b] is not None:
                wd[b].wait()

    return k(*args).reshape(n, d)


def _rms(x, w):
    ms = jnp.mean(jnp.square(x), axis=-1, keepdims=True)
    return x / jnp.sqrt(ms + EPS) * w


def _silu(x):
    return x * jax.nn.sigmoid(x)


def _bdot(a, b, dims):
    return lax.dot_general(a.astype(jnp.bfloat16), b.astype(jnp.bfloat16),
                           (dims, ((), ())),
                           preferred_element_type=jnp.float32)


def _qkv_body(x_ref, w_ref, ln_ref, qn_ref, kn_ref, sin_ref, cos_ref,
              q_ref, k_ref, v_ref):
    h = _rms(x_ref[...], ln_ref[0])
    qkv = lax.dot_general(h, w_ref[...], (((1,), (1,)), ((), ())))
    bt = qkv.shape[0]
    q = qkv[:, :C].reshape(bt, NH, HD)
    k = qkv[:, C:2 * C].reshape(bt, NH, HD)
    v = qkv[:, 2 * C:]
    s = sin_ref[...][:, None, :]
    c = cos_ref[...][:, None, :]

    def rope(z):
        z1 = z[..., :HD // 2]
        z2 = z[..., HD // 2:]
        return jnp.concatenate([z1 * c - z2 * s, z1 * s + z2 * c], axis=-1)

    q = _rms(rope(q), qn_ref[0])
    k = _rms(rope(k), kn_ref[0])
    q_ref[...] = jnp.transpose(q, (1, 0, 2))
    k_ref[...] = jnp.transpose(k, (1, 0, 2))
    v_ref[...] = jnp.transpose(v.reshape(bt, NH, HD), (1, 0, 2))


def _qkv(x, lp, sin, cos):
    t = x.shape[0]
    btok = 256
    grid = (t // btok,)
    return pl.pallas_call(
        _qkv_body,
        grid=grid,
        in_specs=[
            pl.BlockSpec((btok, C), lambda i: (i, 0)),
            pl.BlockSpec((3 * C, C), lambda i: (0, 0)),
            pl.BlockSpec((1, C), lambda i: (0, 0)),
            pl.BlockSpec((1, HD), lambda i: (0, 0)),
            pl.BlockSpec((1, HD), lambda i: (0, 0)),
            pl.BlockSpec((btok, HD // 2), lambda i: (i, 0)),
            pl.BlockSpec((btok, HD // 2), lambda i: (i, 0)),
        ],
        out_specs=[
            pl.BlockSpec((NH, btok, HD), lambda i: (0, i, 0)),
            pl.BlockSpec((NH, btok, HD), lambda i: (0, i, 0)),
            pl.BlockSpec((NH, btok, HD), lambda i: (0, i, 0)),
        ],
        out_shape=[jax.ShapeDtypeStruct((NH, t, HD), jnp.float32)] * 3,
    )(x, lp['attn_w'], lp['ln1_w'].reshape(1, C),
      lp['q_norm_w'].reshape(1, HD), lp['k_norm_w'].reshape(1, HD), sin, cos)


def _attn_body(q_ref, k_ref, v_ref, o_ref):
    qi = pl.program_id(1)
    bq = q_ref.shape[1]
    bk = 256
    q = q_ref[0]

    def step(j, carry):
        acc, m, l = carry
        k = k_ref[0, pl.ds(j * bk, bk), :]
        v = v_ref[0, pl.ds(j * bk, bk), :]
        s = lax.dot_general(q, k, (((1,), (1,)), ((), ()))) * 0.125
        rows = qi * bq + lax.broadcasted_iota(jnp.int32, (bq, bk), 0)
        cols = j * bk + lax.broadcasted_iota(jnp.int32, (bq, bk), 1)
        s = jnp.where(cols <= rows, s, -1e30)
        m_new = jnp.maximum(m, jnp.max(s, axis=-1, keepdims=True))
        alpha = jnp.exp(m - m_new)
        p = jnp.exp(s - m_new)
        l_new = l * alpha + jnp.sum(p, axis=-1, keepdims=True)
        acc_new = acc * alpha + jnp.dot(p, v)
        return acc_new, m_new, l_new

    acc0 = jnp.zeros((bq, HD), jnp.float32)
    m0 = jnp.full((bq, 1), -1e30, jnp.float32)
    l0 = jnp.zeros((bq, 1), jnp.float32)
    nkv = (qi + 1) * (bq // bk)
    acc, m, l = lax.fori_loop(0, nkv, step, (acc0, m0, l0))
    o_ref[0] = acc / l


def _attention(q, k, v):
    t = q.shape[1]
    bq = 256
    grid = (NH, t // bq)
    return pl.pallas_call(
        _attn_body,
        grid=grid,
        in_specs=[
            pl.BlockSpec((1, bq, HD), lambda h, i: (h, i, 0)),
            pl.BlockSpec((1, t, HD), lambda h, i: (h, 0, 0)),
            pl.BlockSpec((1, t, HD), lambda h, i: (h, 0, 0)),
        ],
        out_specs=pl.BlockSpec((1, bq, HD), lambda h, i: (h, i, 0)),
        out_shape=jax.ShapeDtypeStruct((NH, t, HD), jnp.float32),
    )(q, k, v)


def _proj_body(a_ref, w_ref, x_ref, ln_ref, xn_ref, h2_ref):
    bt = a_ref.shape[1]
    am = jnp.transpose(a_ref[...], (1, 0, 2)).reshape(bt, C)
    a = lax.dot_general(am, w_ref[...], (((1,), (1,)), ((), ())))
    xn = x_ref[...] + a
    xn_ref[...] = xn
    h2_ref[...] = _rms(xn, ln_ref[0])


def _proj(attn, x, lp):
    t = x.shape[0]
    btok = 256
    return pl.pallas_call(
        _proj_body,
        grid=(t // btok,),
        in_specs=[
            pl.BlockSpec((NH, btok, HD), lambda i: (0, i, 0)),
            pl.BlockSpec((C, C), lambda i: (0, 0)),
            pl.BlockSpec((btok, C), lambda i: (i, 0)),
            pl.BlockSpec((1, C), lambda i: (0, 0)),
        ],
        out_specs=[
            pl.BlockSpec((btok, C), lambda i: (i, 0)),
            pl.BlockSpec((btok, C), lambda i: (i, 0)),
        ],
        out_shape=[jax.ShapeDtypeStruct((t, C), jnp.float32)] * 2,
    )(attn, lp['proj_w'], x, lp['ln2_w'].reshape(1, C))


def _router_body(h2_ref, gw_ref, gb_ref, ti_ref, wn_ref):
    s = jax.nn.sigmoid(
        lax.dot_general(h2_ref[...], gw_ref[...], (((1,), (1,)), ((), ()))))
    work = s + gb_ref[0]
    bt = s.shape[0]
    iota = lax.broadcasted_iota(jnp.int32, (bt, E), 1)
    idxs = []
    wvals = []
    for _ in range(TOPK):
        m = jnp.max(work, axis=-1, keepdims=True)
        cand = jnp.where(work == m, iota, E)
        ix = jnp.min(cand, axis=-1, keepdims=True)
        hit = iota == ix
        wvals.append(jnp.sum(jnp.where(hit, s, 0.0), axis=-1, keepdims=True))
        idxs.append(ix)
        work = jnp.where(hit, -jnp.float32(1e30), work)
    ti_ref[...] = jnp.concatenate(idxs, axis=-1)
    wv = jnp.concatenate(wvals, axis=-1)
    wn_ref[...] = wv / jnp.sum(wv, axis=-1, keepdims=True)


def _router(h2, lp):
    t = h2.shape[0]
    btok = 256
    return pl.pallas_call(
        _router_body,
        grid=(t // btok,),
        in_specs=[
            pl.BlockSpec((btok, C), lambda i: (i, 0)),
            pl.BlockSpec((E, C), lambda i: (0, 0)),
            pl.BlockSpec((1, E), lambda i: (0, 0)),
        ],
        out_specs=[
            pl.BlockSpec((btok, TOPK), lambda i: (i, 0)),
            pl.BlockSpec((btok, TOPK), lambda i: (i, 0)),
        ],
        out_shape=[
            jax.ShapeDtypeStruct((t, TOPK), jnp.int32),
            jax.ShapeDtypeStruct((t, TOPK), jnp.float32),
        ],
    )(h2, lp['gate_w'], lp['gate_bias'].reshape(1, E))


def _moe_body(be_ref, xs_ref, up_ref, dn_ref, ws_ref, o_ref):
    b = pl.program_id(0)

    @pl.when(b < be_ref[NBLK])
    def _go():
        x = xs_ref[...]
        h = _bdot(x, up_ref[0], ((1,), (1,)))
        hs = _silu(h[:, :EH]) * h[:, EH:]
        d = _bdot(hs, dn_ref[0], ((1,), (1,)))
        o_ref[...] = d * ws_ref[0, 0][:, None]


def _moe_gemm(be, xs, up_w, down_w, ws3):
    grid_spec = pltpu.PrefetchScalarGridSpec(
        num_scalar_prefetch=1,
        grid=(NBLK,),
        in_specs=[
            pl.BlockSpec((BT, C), lambda b, be: (b, 0)),
            pl.BlockSpec((1, 2 * EH, C), lambda b, be: (be[b], 0, 0)),
            pl.BlockSpec((1, C, EH), lambda b, be: (be[b], 0, 0)),
            pl.BlockSpec((1, 1, BT), lambda b, be: (b, 0, 0)),
        ],
        out_specs=pl.BlockSpec((BT, C), lambda b, be: (b, 0)),
    )
    return pl.pallas_call(
        _moe_body,
        grid_spec=grid_spec,
        out_shape=jax.ShapeDtypeStruct((PN, C), jnp.float32),
    )(be, xs, up_w, down_w, ws3)


def _comb_body(op_ref, h2_ref, x_ref, sgw_ref, sdw_ref, o_ref):
    moe = op_ref[:, :C]
    for k in range(1, TOPK):
        moe = moe + op_ref[:, k * C:(k + 1) * C]
    gp = _bdot(h2_ref[...], sgw_ref[...], ((1,), (1,)))
    y = gp[:, :SH_HID]
    g = gp[:, SH_HID:]
    sh = _bdot(_silu(g) * y, sdw_ref[...], ((1,), (1,)))
    o_ref[...] = x_ref[...] + sh + moe


def _combine(op, h2, x, lp):
    t = x.shape[0]
    btok = 256
    return pl.pallas_call(
        _comb_body,
        grid=(t // btok,),
        in_specs=[
            pl.BlockSpec((btok, TOPK * C), lambda i: (i, 0)),
            pl.BlockSpec((btok, C), lambda i: (i, 0)),
            pl.BlockSpec((btok, C), lambda i: (i, 0)),
            pl.BlockSpec((2 * SH_HID, C), lambda i: (0, 0)),
            pl.BlockSpec((C, SH_HID), lambda i: (0, 0)),
        ],
        out_specs=pl.BlockSpec((btok, C), lambda i: (i, 0)),
        out_shape=jax.ShapeDtypeStruct((t, C), jnp.float32),
    )(op, h2, x, lp['shared_gate_w'], lp['shared_down_w'])


def _head_body(x_ref, ln_ref, w_ref, o_ref):
    xn = _rms(x_ref[...], ln_ref[0])
    o_ref[...] = _bdot(xn, w_ref[...], ((1,), (1,)))


def _lm_head(x, ln_w, wte):
    t = x.shape[0]
    btok = 256
    bv = 1024
    return pl.pallas_call(
        _head_body,
        grid=(VOCAB // bv, t // btok),
        in_specs=[
            pl.BlockSpec((btok, C), lambda j, i: (i, 0)),
            pl.BlockSpec((1, C), lambda j, i: (0, 0)),
            pl.BlockSpec((bv, C), lambda j, i: (j, 0)),
        ],
        out_specs=pl.BlockSpec((btok, bv), lambda j, i: (i, j)),
        out_shape=jax.ShapeDtypeStruct((t, VOCAB), jnp.float32),
    )(x, ln_w.reshape(1, C), wte)


def _route_tables(ti, wn):
    n = ti.shape[0] * TOPK
    eids = ti.reshape(-1)
    wflat = wn.reshape(-1)
    counts = jnp.bincount(eids, length=E)
    offs = jnp.cumsum(counts) - counts
    pe = ((counts + BT - 1) // BT) * BT
    pb = jnp.cumsum(pe) - pe
    order = jnp.argsort(eids)
    es = eids[order]
    ppos_sorted = pb[es] + (jnp.arange(n, dtype=jnp.int32) - offs[es])
    ntok = ti.shape[0]
    tok_padded = (jnp.arange(PN, dtype=jnp.int32) % ntok).at[ppos_sorted].set(
        (order // TOPK).astype(jnp.int32))
    ws_padded = jnp.zeros((PN,), jnp.float32).at[ppos_sorted].set(wflat[order])
    ppos = jnp.zeros((n,), jnp.int32).at[order].set(
        ppos_sorted.astype(jnp.int32))
    nb = pe // BT
    be = jnp.repeat(jnp.arange(E, dtype=jnp.int32), nb,
                    total_repeat_length=NBLK)
    nvb = jnp.sum(nb, dtype=jnp.int32)
    be = jnp.concatenate([be, nvb[None]])
    nv16 = jnp.broadcast_to(nvb * BT, (16,)).astype(jnp.int32)
    return tok_padded, ws_padded.reshape(NBLK, 1, BT), ppos, be, nv16


def _layer(x, lp, sin, cos):
    q, k, v = _qkv(x, lp, sin, cos)
    attn = _attention(q, k, v)
    xn, h2 = _proj(attn, x, lp)
    ti, wn = _router(h2, lp)
    tok_padded, ws3, ppos, be, nv16 = _route_tables(ti, wn)
    xs = _sc_gather(h2, tok_padded)
    out_sorted = _moe_gemm(be, xs, lp['up_w'], lp['down_w'], ws3)
    op = _sc_gather(out_sorted, ppos)
    t = x.shape[0]
    return _combine(op.reshape(t, TOPK * C), h2, xn, lp)


def kernel(params, idx):
    t = idx.shape[1]
    ids = idx.reshape(-1).astype(jnp.int32)
    x = _sc_gather(params['wte'], ids)
    inv = 1.0 / (10000.0 ** (jnp.arange(0, HD, 2, dtype=jnp.float32) / HD))
    ang = jnp.arange(t, dtype=jnp.float32)[:, None] * inv[None, :]
    sin, cos = jnp.sin(ang), jnp.cos(ang)
    for lp in params['blocks']:
        x = _layer(x, lp, sin, cos)
    logits = _lm_head(x, params['ln_w'], params['wte'])
    return logits.reshape(1, t, VOCAB)
